# baseline (speedup 1.0000x reference)
import functools

import jax
import jax.numpy as jnp
from jax import lax
from jax.experimental import pallas as pl
from jax.experimental.pallas import tpu as pltpu
from jax.experimental.pallas import tpu_sc as plsc

DIM = 64
NC, NS = 2, 16
NW = NC * NS


@functools.partial(jax.jit, static_argnames=("b", "s"))
def _sc_gather(x, emb, b, s):
    bpw = b // NW
    n_half = bpw // 2
    s0 = min(128, s)
    s1 = s - s0
    mesh = plsc.VectorSubcoreMesh(core_axis_name="c", subcore_axis_name="s")

    @functools.partial(
        pl.kernel,
        out_type=jax.ShapeDtypeStruct((b, s, DIM), jnp.float32),
        mesh=mesh,
        compiler_params=pltpu.CompilerParams(use_tc_tiling_on_sc=False),
        scratch_types=[
            pltpu.VMEM((bpw, s), jnp.int32),
            pltpu.VMEM((2, s, DIM), jnp.float32),
            pltpu.SemaphoreType.DMA,
            pltpu.SemaphoreType.DMA,
            pltpu.SemaphoreType.DMA,
            pltpu.SemaphoreType.DMA,
        ],
    )
    def k(x_hbm, emb_hbm, out_hbm, idx_all, rows_v, gsem0, gsem1, osem0, osem1):
        wid = lax.axis_index("s") * NC + lax.axis_index("c")
        base = wid * bpw
        pltpu.sync_copy(x_hbm.at[pl.ds(base, bpw)], idx_all)

        r0 = rows_v.at[0]
        r1 = rows_v.at[1]

        def fire_gathers(i, buf, sem):
            pltpu.async_copy(
                emb_hbm.at[idx_all.at[i, pl.ds(0, s0)]], buf.at[pl.ds(0, s0)], sem)
            if s1:
                pltpu.async_copy(
                    emb_hbm.at[idx_all.at[i, pl.ds(s0, s1)]], buf.at[pl.ds(s0, s1)], sem)

        def fire_store(i, buf, sem):
            pltpu.async_copy(buf, out_hbm.at[base + i], sem)

        def wait_bytes(buf, sem):
            pltpu.make_async_copy(buf, out_hbm.at[base], sem).wait()

        fire_gathers(0, r0, gsem0)

        def body2(t, carry):
            i0 = 2 * t

            @pl.when(t > 0)
            def _():
                wait_bytes(r1, osem1)
            fire_gathers(i0 + 1, r1, gsem1)
            wait_bytes(r0, gsem0)
            fire_store(i0, r0, osem0)

            @pl.when(t + 1 < n_half)
            def _():
                wait_bytes(r0, osem0)
                fire_gathers(i0 + 2, r0, gsem0)
            wait_bytes(r1, gsem1)
            fire_store(i0 + 1, r1, osem1)
            return carry

        lax.fori_loop(0, n_half, body2, 0)
        wait_bytes(r0, osem0)
        wait_bytes(r1, osem1)

    return k(x, emb)


def kernel(x, emb):
    b, s = x.shape
    return _sc_gather(x.astype(jnp.int32), emb, b, s)
